# Initial kernel scaffold; baseline (speedup 1.0000x reference)
import functools

import jax
import jax.numpy as jnp
from jax import lax
from jax.experimental import pallas as pl
from jax.experimental.pallas import tpu as pltpu
from jax.experimental.pallas import tpu_sc as plsc

N = 10000
E = 320000
D = 128

NC = 2
NS = 16
NW = NC * NS
EPW = E // NW
K = 128
NFULL = EPW // K
TAIL = EPW - NFULL * K
RPS = N // NS
ZR = 125

BLK = 400
NBLK = N // BLK

_MESH = plsc.VectorSubcoreMesh(core_axis_name="c", subcore_axis_name="s")


@functools.partial(
    pl.kernel,
    out_type=jax.ShapeDtypeStruct((NC, 2, N, 16), jnp.float32),
    mesh=_MESH,
    scratch_types=[
        pltpu.VMEM_SHARED((N, 16), jnp.float32),
        pltpu.VMEM_SHARED((N, 16), jnp.float32),
        pltpu.VMEM((K,), jnp.int32),
        pltpu.VMEM((K,), jnp.int32),
        pltpu.VMEM((TAIL,), jnp.int32),
        pltpu.VMEM((TAIL,), jnp.int32),
        pltpu.VMEM((K, 16), jnp.float32),
        pltpu.VMEM((ZR, 16), jnp.float32),
    ],
)
def _deg(src_hbm, dst_hbm, out_hbm, acc_s, acc_d, idx_s, idx_d, idx_st,
         idx_dt, ones_v, zbuf):
    c = lax.axis_index("c")
    s = lax.axis_index("s")
    wid = s * NC + c

    ones16 = jnp.full((16,), 1.0, jnp.float32)
    zero16 = jnp.zeros((16,), jnp.float32)

    @pl.loop(0, K)
    def _(r):
        ones_v[r, :] = ones16

    @pl.loop(0, ZR)
    def _(r):
        zbuf[r, :] = zero16

    @pl.loop(0, RPS // ZR)
    def _(t):
        pltpu.sync_copy(zbuf, acc_s.at[pl.ds(s * RPS + t * ZR, ZR)])
        pltpu.sync_copy(zbuf, acc_d.at[pl.ds(s * RPS + t * ZR, ZR)])

    plsc.subcore_barrier()

    base = wid * EPW

    @pl.loop(0, NFULL)
    def _(j):
        off = base + j * K
        pltpu.sync_copy(src_hbm.at[pl.ds(off, K)], idx_s)
        pltpu.sync_copy(dst_hbm.at[pl.ds(off, K)], idx_d)
        pltpu.sync_copy(ones_v, acc_s.at[idx_s], add=True)
        pltpu.sync_copy(ones_v, acc_d.at[idx_d], add=True)

    toff = base + NFULL * K
    pltpu.sync_copy(src_hbm.at[pl.ds(toff, TAIL)], idx_st)
    pltpu.sync_copy(dst_hbm.at[pl.ds(toff, TAIL)], idx_dt)
    pltpu.sync_copy(ones_v.at[pl.ds(0, TAIL)], acc_s.at[idx_st], add=True)
    pltpu.sync_copy(ones_v.at[pl.ds(0, TAIL)], acc_d.at[idx_dt], add=True)

    plsc.subcore_barrier()

    pltpu.sync_copy(acc_s.at[pl.ds(s * RPS, RPS)],
                    out_hbm.at[c, 0, pl.ds(s * RPS, RPS)])
    pltpu.sync_copy(acc_d.at[pl.ds(s * RPS, RPS)],
                    out_hbm.at[c, 1, pl.ds(s * RPS, RPS)])


@functools.partial(
    pl.kernel,
    out_type=jax.ShapeDtypeStruct((NC, N, D), jnp.float32),
    mesh=_MESH,
    scratch_types=[
        pltpu.VMEM_SHARED((N, D), jnp.float32),
        pltpu.VMEM((K,), jnp.int32),
        pltpu.VMEM((K,), jnp.int32),
        pltpu.VMEM((TAIL,), jnp.int32),
        pltpu.VMEM((TAIL,), jnp.int32),
        pltpu.VMEM((K, D), jnp.float32),
        pltpu.VMEM((TAIL, D), jnp.float32),
        pltpu.VMEM((ZR, D), jnp.float32),
        pltpu.SemaphoreType.DMA,
    ],
)
def _agg(h_hbm, src_hbm, dst_hbm, out_hbm, acc, idx_s, idx_d, idx_st, idx_dt,
         rows, rows_t, zbuf, sem):
    c = lax.axis_index("c")
    s = lax.axis_index("s")
    wid = s * NC + c

    zero16 = jnp.zeros((16,), jnp.float32)

    @pl.loop(0, ZR)
    def _(r):
        @pl.loop(0, D, step=16)
        def _(col):
            zbuf[r, pl.ds(col, 16)] = zero16

    @pl.loop(0, RPS // ZR)
    def _(t):
        pltpu.sync_copy(zbuf, acc.at[pl.ds(s * RPS + t * ZR, ZR)])

    plsc.subcore_barrier()

    base = wid * EPW

    @pl.loop(0, NFULL)
    def _(j):
        off = base + j * K
        pltpu.sync_copy(src_hbm.at[pl.ds(off, K)], idx_s)
        pltpu.sync_copy(dst_hbm.at[pl.ds(off, K)], idx_d)
        pltpu.async_copy(h_hbm.at[idx_s], rows, sem).wait()
        pltpu.sync_copy(rows, acc.at[idx_d], add=True)

    toff = base + NFULL * K
    pltpu.sync_copy(src_hbm.at[pl.ds(toff, TAIL)], idx_st)
    pltpu.sync_copy(dst_hbm.at[pl.ds(toff, TAIL)], idx_dt)
    pltpu.async_copy(h_hbm.at[idx_st], rows_t, sem).wait()
    pltpu.sync_copy(rows_t, acc.at[idx_dt], add=True)

    plsc.subcore_barrier()

    pltpu.sync_copy(acc.at[pl.ds(s * RPS, RPS)],
                    out_hbm.at[c, pl.ds(s * RPS, RPS)])


def _prep_body(degp_ref, feat_ref, h0s_ref, din_ref, dout_ref):
    dsrc = degp_ref[0, 0] + degp_ref[1, 0]
    ddst = degp_ref[0, 1] + degp_ref[1, 1]
    dsv = jnp.max(dsrc, axis=1, keepdims=True)
    ddv = jnp.max(ddst, axis=1, keepdims=True)
    iso = lax.rsqrt(jnp.maximum(dsv, 1.0))
    isi = lax.rsqrt(jnp.maximum(ddv, 1.0))
    dout_ref[...] = jnp.broadcast_to(iso, (BLK, D))
    din_ref[...] = jnp.broadcast_to(isi, (BLK, D))
    h0s_ref[...] = feat_ref[...] * iso


def _prep(degp, feature):
    return pl.pallas_call(
        _prep_body,
        grid=(NBLK,),
        in_specs=[
            pl.BlockSpec((NC, 2, BLK, 16), lambda i: (0, 0, i, 0)),
            pl.BlockSpec((BLK, D), lambda i: (i, 0)),
        ],
        out_specs=[
            pl.BlockSpec((BLK, D), lambda i: (i, 0)),
            pl.BlockSpec((BLK, D), lambda i: (i, 0)),
            pl.BlockSpec((BLK, D), lambda i: (i, 0)),
        ],
        out_shape=[
            jax.ShapeDtypeStruct((N, D), jnp.float32),
            jax.ShapeDtypeStruct((N, D), jnp.float32),
            jax.ShapeDtypeStruct((N, D), jnp.float32),
        ],
    )(degp, feature)


def _layer_body(m_ref, din_ref, dout_ref, w_ref, b_ref, out_ref):
    m = (m_ref[0] + m_ref[1]) * din_ref[...]
    h = lax.dot_general(m, w_ref[...], (((1,), (0,)), ((), ())),
                        precision=lax.Precision.HIGHEST,
                        preferred_element_type=jnp.float32)
    h = jnp.maximum(h + b_ref[...], 0.0)
    out_ref[...] = h * dout_ref[...]


def _layer(m_parts, din_b, dout_b, w, b2d):
    return pl.pallas_call(
        _layer_body,
        grid=(NBLK,),
        in_specs=[
            pl.BlockSpec((NC, BLK, D), lambda i: (0, i, 0)),
            pl.BlockSpec((BLK, D), lambda i: (i, 0)),
            pl.BlockSpec((BLK, D), lambda i: (i, 0)),
            pl.BlockSpec((D, D), lambda i: (0, 0)),
            pl.BlockSpec((1, D), lambda i: (0, 0)),
        ],
        out_specs=pl.BlockSpec((BLK, D), lambda i: (i, 0)),
        out_shape=jax.ShapeDtypeStruct((N, D), jnp.float32),
    )(m_parts, din_b, dout_b, w, b2d)


def _final_body(m_ref, din_ref, w_ref, b_ref, h_ref, hg_ref):
    m = (m_ref[0] + m_ref[1]) * din_ref[...]
    h = lax.dot_general(m, w_ref[...], (((1,), (0,)), ((), ())),
                        precision=lax.Precision.HIGHEST,
                        preferred_element_type=jnp.float32)
    h = jnp.maximum(h + b_ref[...], 0.0)
    h_ref[...] = h

    @pl.when(pl.program_id(0) == 0)
    def _():
        hg_ref[...] = jnp.zeros((1, D), jnp.float32)

    hg_ref[...] += jnp.sum(h, axis=0, keepdims=True) * (1.0 / N)


def _final(m_parts, din_b, w, b2d):
    return pl.pallas_call(
        _final_body,
        grid=(NBLK,),
        in_specs=[
            pl.BlockSpec((NC, BLK, D), lambda i: (0, i, 0)),
            pl.BlockSpec((BLK, D), lambda i: (i, 0)),
            pl.BlockSpec((D, D), lambda i: (0, 0)),
            pl.BlockSpec((1, D), lambda i: (0, 0)),
        ],
        out_specs=[
            pl.BlockSpec((BLK, D), lambda i: (i, 0)),
            pl.BlockSpec((1, D), lambda i: (0, 0)),
        ],
        out_shape=[
            jax.ShapeDtypeStruct((N, D), jnp.float32),
            jax.ShapeDtypeStruct((1, D), jnp.float32),
        ],
    )(m_parts, din_b, w, b2d)


def kernel(feature, edge_index, W0, b0, W1, b1, W2, b2):
    src = edge_index[0]
    dst = edge_index[1]

    degp = _deg(src, dst)
    h0s, din_b, dout_b = _prep(degp, feature)

    m0 = _agg(h0s, src, dst)
    h1s = _layer(m0, din_b, dout_b, W0, b0.reshape(1, D))
    m1 = _agg(h1s, din_b, dout_b, W0, b0.reshape(1, D)) if False else _agg(h1s, src, dst)
    h2s = _layer(m1, din_b, dout_b, W1, b1.reshape(1, D))
    m2 = _agg(h2s, src, dst)
    h, hg = _final(m2, din_b, W2, b2.reshape(1, D))
    return (h, hg)


# speedup vs baseline: 3.8410x; 3.8410x over previous
import dataclasses
import functools

import jax
import jax.numpy as jnp
from jax import lax
from jax.experimental import pallas as pl
from jax.experimental.pallas import tpu as pltpu
from jax.experimental.pallas import tpu_sc as plsc

N = 10000
E = 320000
D = 128

NC = 2
NS = 16
NW = NC * NS
EPW = E // NW
K = 128
NFULL = EPW // K
TAIL = EPW - NFULL * K
RPS = N // NS
ZR = 25
WB = 624
CH = 208

BLK = 400
NBLK = N // BLK

_MESH = plsc.VectorSubcoreMesh(core_axis_name="c", subcore_axis_name="s")

_SC_PARAMS = pltpu.CompilerParams()
if "needs_layout_passes" in pltpu.CompilerParams.__dataclass_fields__:
    _SC_PARAMS = dataclasses.replace(_SC_PARAMS, needs_layout_passes=False)


@functools.partial(
    pl.kernel,
    out_type=jax.ShapeDtypeStruct((2 * NW * N,), jnp.float32),
    mesh=_MESH,
    scratch_types=[
        pltpu.VMEM((N,), jnp.float32),
        pltpu.VMEM((N,), jnp.float32),
        pltpu.VMEM((K,), jnp.int32),
        pltpu.VMEM((K,), jnp.int32),
    ],
    compiler_params=_SC_PARAMS,
)
def _deg(src_hbm, dst_hbm, out_hbm, acc_s, acc_d, idx_s, idx_d):
    c = lax.axis_index("c")
    s = lax.axis_index("s")
    wid = s * NC + c

    ones16 = jnp.full((16,), 1.0, jnp.float32)
    zero16 = jnp.zeros((16,), jnp.float32)

    @pl.loop(0, N // 16)
    def _(r):
        acc_s[pl.ds(r * 16, 16)] = zero16
        acc_d[pl.ds(r * 16, 16)] = zero16

    base = wid * EPW

    @pl.loop(0, NFULL)
    def _(j):
        off = base + j * K
        pltpu.sync_copy(src_hbm.at[pl.ds(off, K)], idx_s)
        pltpu.sync_copy(dst_hbm.at[pl.ds(off, K)], idx_d)

        @pl.loop(0, K // 16)
        def _(q):
            plsc.addupdate_scatter(acc_s, [idx_s[pl.ds(q * 16, 16)]], ones16)
            plsc.addupdate_scatter(acc_d, [idx_d[pl.ds(q * 16, 16)]], ones16)

    toff = base + NFULL * K
    pltpu.sync_copy(src_hbm.at[pl.ds(toff, TAIL)], idx_s.at[pl.ds(0, TAIL)])
    pltpu.sync_copy(dst_hbm.at[pl.ds(toff, TAIL)], idx_d.at[pl.ds(0, TAIL)])
    plsc.addupdate_scatter(acc_s, [idx_s[pl.ds(0, TAIL)]], ones16)
    plsc.addupdate_scatter(acc_d, [idx_d[pl.ds(0, TAIL)]], ones16)

    pltpu.sync_copy(acc_s, out_hbm.at[pl.ds(wid * N, N)])
    pltpu.sync_copy(acc_d, out_hbm.at[pl.ds(NW * N + wid * N, N)])


@functools.partial(
    pl.kernel,
    out_type=jax.ShapeDtypeStruct((NC, N, D), jnp.float32),
    mesh=_MESH,
    scratch_types=[
        pltpu.VMEM_SHARED((N, D), jnp.float32),
        pltpu.VMEM((K,), jnp.int32),
        pltpu.VMEM((K,), jnp.int32),
        pltpu.VMEM((TAIL,), jnp.int32),
        pltpu.VMEM((TAIL,), jnp.int32),
        pltpu.VMEM((K, D), jnp.float32),
        pltpu.VMEM((ZR, D), jnp.float32),
        pltpu.SemaphoreType.DMA,
    ],
)
def _agg(h_hbm, src_hbm, dst_hbm, out_hbm, acc, idx_s, idx_d, idx_st, idx_dt,
         rows, zbuf, sem):
    c = lax.axis_index("c")
    s = lax.axis_index("s")
    wid = s * NC + c

    zero16 = jnp.zeros((16,), jnp.float32)

    @pl.loop(0, ZR)
    def _(r):
        @pl.loop(0, D, step=16)
        def _(col):
            zbuf[r, pl.ds(col, 16)] = zero16

    @pl.loop(0, RPS // ZR)
    def _(t):
        pltpu.sync_copy(zbuf, acc.at[pl.ds(s * RPS + t * ZR, ZR)])

    plsc.subcore_barrier()

    base = wid * EPW

    @pl.loop(0, NFULL)
    def _(j):
        off = base + j * K
        pltpu.sync_copy(src_hbm.at[pl.ds(off, K)], idx_s)
        pltpu.sync_copy(dst_hbm.at[pl.ds(off, K)], idx_d)
        pltpu.async_copy(h_hbm.at[idx_s], rows, sem).wait()
        pltpu.sync_copy(rows, acc.at[idx_d], add=True)

    toff = base + NFULL * K
    pltpu.sync_copy(src_hbm.at[pl.ds(toff, TAIL)], idx_st)
    pltpu.sync_copy(dst_hbm.at[pl.ds(toff, TAIL)], idx_dt)
    pltpu.async_copy(h_hbm.at[idx_st], rows.at[pl.ds(0, TAIL)], sem).wait()
    pltpu.sync_copy(rows.at[pl.ds(0, TAIL)], acc.at[idx_dt], add=True)

    plsc.subcore_barrier()

    pltpu.sync_copy(acc.at[pl.ds(s * WB, WB)],
                    out_hbm.at[c, pl.ds(s * WB, WB)])

    @pl.when(s == NS - 1)
    def _():
        pltpu.sync_copy(acc.at[pl.ds(NS * WB, N - NS * WB)],
                        out_hbm.at[c, pl.ds(NS * WB, N - NS * WB)])


def _prep_body(degp_ref, feat_ref, h0s_ref, din_ref, dout_ref):
    dsv = jnp.sum(degp_ref[0], axis=0)
    ddv = jnp.sum(degp_ref[1], axis=0)
    iso = lax.rsqrt(jnp.maximum(dsv, 1.0))
    isi = lax.rsqrt(jnp.maximum(ddv, 1.0))
    dout_ref[...] = jnp.broadcast_to(iso, (BLK, D))
    din_ref[...] = jnp.broadcast_to(isi, (BLK, D))
    h0s_ref[...] = feat_ref[...] * iso


def _prep(degp, feature):
    return pl.pallas_call(
        _prep_body,
        grid=(NBLK,),
        in_specs=[
            pl.BlockSpec((2, NW, BLK, 1), lambda i: (0, 0, i, 0)),
            pl.BlockSpec((BLK, D), lambda i: (i, 0)),
        ],
        out_specs=[
            pl.BlockSpec((BLK, D), lambda i: (i, 0)),
            pl.BlockSpec((BLK, D), lambda i: (i, 0)),
            pl.BlockSpec((BLK, D), lambda i: (i, 0)),
        ],
        out_shape=[
            jax.ShapeDtypeStruct((N, D), jnp.float32),
            jax.ShapeDtypeStruct((N, D), jnp.float32),
            jax.ShapeDtypeStruct((N, D), jnp.float32),
        ],
    )(degp, feature)


def _layer_body(m_ref, din_ref, dout_ref, w_ref, b_ref, out_ref):
    m = (m_ref[0] + m_ref[1]) * din_ref[...]
    h = lax.dot_general(m, w_ref[...], (((1,), (0,)), ((), ())),
                        precision=lax.Precision.HIGHEST,
                        preferred_element_type=jnp.float32)
    h = jnp.maximum(h + b_ref[...], 0.0)
    out_ref[...] = h * dout_ref[...]


def _layer(m_parts, din_b, dout_b, w, b2d):
    return pl.pallas_call(
        _layer_body,
        grid=(NBLK,),
        in_specs=[
            pl.BlockSpec((NC, BLK, D), lambda i: (0, i, 0)),
            pl.BlockSpec((BLK, D), lambda i: (i, 0)),
            pl.BlockSpec((BLK, D), lambda i: (i, 0)),
            pl.BlockSpec((D, D), lambda i: (0, 0)),
            pl.BlockSpec((1, D), lambda i: (0, 0)),
        ],
        out_specs=pl.BlockSpec((BLK, D), lambda i: (i, 0)),
        out_shape=jax.ShapeDtypeStruct((N, D), jnp.float32),
    )(m_parts, din_b, dout_b, w, b2d)


def _final_body(m_ref, din_ref, w_ref, b_ref, h_ref, hg_ref):
    m = (m_ref[0] + m_ref[1]) * din_ref[...]
    h = lax.dot_general(m, w_ref[...], (((1,), (0,)), ((), ())),
                        precision=lax.Precision.HIGHEST,
                        preferred_element_type=jnp.float32)
    h = jnp.maximum(h + b_ref[...], 0.0)
    h_ref[...] = h

    @pl.when(pl.program_id(0) == 0)
    def _():
        hg_ref[...] = jnp.zeros((1, D), jnp.float32)

    hg_ref[...] += jnp.sum(h, axis=0, keepdims=True) * (1.0 / N)


def _final(m_parts, din_b, w, b2d):
    return pl.pallas_call(
        _final_body,
        grid=(NBLK,),
        in_specs=[
            pl.BlockSpec((NC, BLK, D), lambda i: (0, i, 0)),
            pl.BlockSpec((BLK, D), lambda i: (i, 0)),
            pl.BlockSpec((D, D), lambda i: (0, 0)),
            pl.BlockSpec((1, D), lambda i: (0, 0)),
        ],
        out_specs=[
            pl.BlockSpec((BLK, D), lambda i: (i, 0)),
            pl.BlockSpec((1, D), lambda i: (0, 0)),
        ],
        out_shape=[
            jax.ShapeDtypeStruct((N, D), jnp.float32),
            jax.ShapeDtypeStruct((1, D), jnp.float32),
        ],
    )(m_parts, din_b, w, b2d)


def kernel(feature, edge_index, W0, b0, W1, b1, W2, b2):
    src = edge_index[0]
    dst = edge_index[1]

    degp = _deg(src, dst).reshape(2, NW, N, 1)
    h0s, din_b, dout_b = _prep(degp, feature)

    m0 = _agg(h0s, src, dst)
    h1s = _layer(m0, din_b, dout_b, W0, b0.reshape(1, D))
    m1 = _agg(h1s, src, dst)
    h2s = _layer(m1, din_b, dout_b, W1, b1.reshape(1, D))
    m2 = _agg(h2s, src, dst)
    h, hg = _final(m2, din_b, W2, b2.reshape(1, D))
    return (h, hg)


# speedup vs baseline: 4.4409x; 1.1562x over previous
import dataclasses
import functools

import jax
import jax.numpy as jnp
from jax import lax
from jax.experimental import pallas as pl
from jax.experimental.pallas import tpu as pltpu
from jax.experimental.pallas import tpu_sc as plsc

N = 10000
E = 320000
D = 128

NC = 2
NS = 16
NW = NC * NS
EPW = E // NW
K = 128
NFULL = EPW // K
TAIL = EPW - NFULL * K
KA = 80
NCH = EPW // KA
RPS = N // NS
ZR = 25
WB = 624
CH = 208

BLK = 400
NBLK = N // BLK

_MESH = plsc.VectorSubcoreMesh(core_axis_name="c", subcore_axis_name="s")

_SC_PARAMS = pltpu.CompilerParams()
if "needs_layout_passes" in pltpu.CompilerParams.__dataclass_fields__:
    _SC_PARAMS = dataclasses.replace(_SC_PARAMS, needs_layout_passes=False)


@functools.partial(
    pl.kernel,
    out_type=jax.ShapeDtypeStruct((2 * NW * N,), jnp.float32),
    mesh=_MESH,
    scratch_types=[
        pltpu.VMEM((N,), jnp.float32),
        pltpu.VMEM((N,), jnp.float32),
        pltpu.VMEM((K,), jnp.int32),
        pltpu.VMEM((K,), jnp.int32),
    ],
    compiler_params=_SC_PARAMS,
)
def _deg(src_hbm, dst_hbm, out_hbm, acc_s, acc_d, idx_s, idx_d):
    c = lax.axis_index("c")
    s = lax.axis_index("s")
    wid = s * NC + c

    ones16 = jnp.full((16,), 1.0, jnp.float32)
    zero16 = jnp.zeros((16,), jnp.float32)

    @pl.loop(0, N // 16)
    def _(r):
        acc_s[pl.ds(r * 16, 16)] = zero16
        acc_d[pl.ds(r * 16, 16)] = zero16

    base = wid * EPW

    @pl.loop(0, NFULL)
    def _(j):
        off = base + j * K
        pltpu.sync_copy(src_hbm.at[pl.ds(off, K)], idx_s)
        pltpu.sync_copy(dst_hbm.at[pl.ds(off, K)], idx_d)

        @pl.loop(0, K // 16)
        def _(q):
            plsc.addupdate_scatter(acc_s, [idx_s[pl.ds(q * 16, 16)]], ones16)
            plsc.addupdate_scatter(acc_d, [idx_d[pl.ds(q * 16, 16)]], ones16)

    toff = base + NFULL * K
    pltpu.sync_copy(src_hbm.at[pl.ds(toff, TAIL)], idx_s.at[pl.ds(0, TAIL)])
    pltpu.sync_copy(dst_hbm.at[pl.ds(toff, TAIL)], idx_d.at[pl.ds(0, TAIL)])
    plsc.addupdate_scatter(acc_s, [idx_s[pl.ds(0, TAIL)]], ones16)
    plsc.addupdate_scatter(acc_d, [idx_d[pl.ds(0, TAIL)]], ones16)

    pltpu.sync_copy(acc_s, out_hbm.at[pl.ds(wid * N, N)])
    pltpu.sync_copy(acc_d, out_hbm.at[pl.ds(NW * N + wid * N, N)])


@functools.partial(
    pl.kernel,
    out_type=jax.ShapeDtypeStruct((NC, N, D), jnp.float32),
    mesh=_MESH,
    scratch_types=[
        pltpu.VMEM_SHARED((N, D), jnp.float32),
        pltpu.VMEM((KA,), jnp.int32),
        pltpu.VMEM((KA,), jnp.int32),
        pltpu.VMEM((KA,), jnp.int32),
        pltpu.VMEM((KA,), jnp.int32),
        pltpu.VMEM((KA, D), jnp.float32),
        pltpu.VMEM((KA, D), jnp.float32),
        pltpu.VMEM((ZR, D), jnp.float32),
        pltpu.SemaphoreType.DMA,
        pltpu.SemaphoreType.DMA,
    ],
)
def _agg(h_hbm, src_hbm, dst_hbm, out_hbm, acc, idx_sa, idx_da, idx_sb,
         idx_db, rows_a, rows_b, zbuf, sem_a, sem_b):
    c = lax.axis_index("c")
    s = lax.axis_index("s")
    wid = s * NC + c

    zero16 = jnp.zeros((16,), jnp.float32)

    @pl.loop(0, ZR)
    def _(r):
        @pl.loop(0, D, step=16)
        def _(col):
            zbuf[r, pl.ds(col, 16)] = zero16

    @pl.loop(0, RPS // ZR)
    def _(t):
        pltpu.sync_copy(zbuf, acc.at[pl.ds(s * RPS + t * ZR, ZR)])

    plsc.subcore_barrier()

    base = wid * EPW

    def load_idx(j, idx_s, idx_d):
        off = base + j * KA
        pltpu.sync_copy(src_hbm.at[pl.ds(off, KA)], idx_s)
        pltpu.sync_copy(dst_hbm.at[pl.ds(off, KA)], idx_d)

    load_idx(0, idx_sa, idx_da)
    ga = pltpu.async_copy(h_hbm.at[idx_sa], rows_a, sem_a)
    load_idx(1, idx_sb, idx_db)
    gb = pltpu.async_copy(h_hbm.at[idx_sb], rows_b, sem_b)

    @pl.loop(0, (NCH - 3) // 2)
    def _(jj):
        j0 = 2 * jj
        pltpu.make_async_copy(h_hbm.at[idx_sa], rows_a, sem_a).wait()
        pltpu.sync_copy(rows_a, acc.at[idx_da], add=True)
        load_idx(j0 + 2, idx_sa, idx_da)
        pltpu.async_copy(h_hbm.at[idx_sa], rows_a, sem_a)
        pltpu.make_async_copy(h_hbm.at[idx_sb], rows_b, sem_b).wait()
        pltpu.sync_copy(rows_b, acc.at[idx_db], add=True)
        load_idx(j0 + 3, idx_sb, idx_db)
        pltpu.async_copy(h_hbm.at[idx_sb], rows_b, sem_b)

    pltpu.make_async_copy(h_hbm.at[idx_sa], rows_a, sem_a).wait()
    pltpu.sync_copy(rows_a, acc.at[idx_da], add=True)
    load_idx(NCH - 1, idx_sa, idx_da)
    pltpu.async_copy(h_hbm.at[idx_sa], rows_a, sem_a)
    pltpu.make_async_copy(h_hbm.at[idx_sb], rows_b, sem_b).wait()
    pltpu.sync_copy(rows_b, acc.at[idx_db], add=True)
    pltpu.make_async_copy(h_hbm.at[idx_sa], rows_a, sem_a).wait()
    pltpu.sync_copy(rows_a, acc.at[idx_da], add=True)

    plsc.subcore_barrier()

    pltpu.sync_copy(acc.at[pl.ds(s * WB, WB)],
                    out_hbm.at[c, pl.ds(s * WB, WB)])

    @pl.when(s == NS - 1)
    def _():
        pltpu.sync_copy(acc.at[pl.ds(NS * WB, N - NS * WB)],
                        out_hbm.at[c, pl.ds(NS * WB, N - NS * WB)])


def _prep_body(degp_ref, feat_ref, h0s_ref, din_ref, dout_ref):
    dsv = jnp.sum(degp_ref[0], axis=0)
    ddv = jnp.sum(degp_ref[1], axis=0)
    iso = lax.rsqrt(jnp.maximum(dsv, 1.0))
    isi = lax.rsqrt(jnp.maximum(ddv, 1.0))
    dout_ref[...] = jnp.broadcast_to(iso, (BLK, D))
    din_ref[...] = jnp.broadcast_to(isi, (BLK, D))
    h0s_ref[...] = feat_ref[...] * iso


def _prep(degp, feature):
    return pl.pallas_call(
        _prep_body,
        grid=(NBLK,),
        in_specs=[
            pl.BlockSpec((2, NW, BLK, 1), lambda i: (0, 0, i, 0)),
            pl.BlockSpec((BLK, D), lambda i: (i, 0)),
        ],
        out_specs=[
            pl.BlockSpec((BLK, D), lambda i: (i, 0)),
            pl.BlockSpec((BLK, D), lambda i: (i, 0)),
            pl.BlockSpec((BLK, D), lambda i: (i, 0)),
        ],
        out_shape=[
            jax.ShapeDtypeStruct((N, D), jnp.float32),
            jax.ShapeDtypeStruct((N, D), jnp.float32),
            jax.ShapeDtypeStruct((N, D), jnp.float32),
        ],
    )(degp, feature)


def _layer_body(m_ref, din_ref, dout_ref, w_ref, b_ref, out_ref):
    m = (m_ref[0] + m_ref[1]) * din_ref[...]
    h = lax.dot_general(m, w_ref[...], (((1,), (0,)), ((), ())),
                        precision=lax.Precision.HIGHEST,
                        preferred_element_type=jnp.float32)
    h = jnp.maximum(h + b_ref[...], 0.0)
    out_ref[...] = h * dout_ref[...]


def _layer(m_parts, din_b, dout_b, w, b2d):
    return pl.pallas_call(
        _layer_body,
        grid=(NBLK,),
        in_specs=[
            pl.BlockSpec((NC, BLK, D), lambda i: (0, i, 0)),
            pl.BlockSpec((BLK, D), lambda i: (i, 0)),
            pl.BlockSpec((BLK, D), lambda i: (i, 0)),
            pl.BlockSpec((D, D), lambda i: (0, 0)),
            pl.BlockSpec((1, D), lambda i: (0, 0)),
        ],
        out_specs=pl.BlockSpec((BLK, D), lambda i: (i, 0)),
        out_shape=jax.ShapeDtypeStruct((N, D), jnp.float32),
    )(m_parts, din_b, dout_b, w, b2d)


def _final_body(m_ref, din_ref, w_ref, b_ref, h_ref, hg_ref):
    m = (m_ref[0] + m_ref[1]) * din_ref[...]
    h = lax.dot_general(m, w_ref[...], (((1,), (0,)), ((), ())),
                        precision=lax.Precision.HIGHEST,
                        preferred_element_type=jnp.float32)
    h = jnp.maximum(h + b_ref[...], 0.0)
    h_ref[...] = h

    @pl.when(pl.program_id(0) == 0)
    def _():
        hg_ref[...] = jnp.zeros((1, D), jnp.float32)

    hg_ref[...] += jnp.sum(h, axis=0, keepdims=True) * (1.0 / N)


def _final(m_parts, din_b, w, b2d):
    return pl.pallas_call(
        _final_body,
        grid=(NBLK,),
        in_specs=[
            pl.BlockSpec((NC, BLK, D), lambda i: (0, i, 0)),
            pl.BlockSpec((BLK, D), lambda i: (i, 0)),
            pl.BlockSpec((D, D), lambda i: (0, 0)),
            pl.BlockSpec((1, D), lambda i: (0, 0)),
        ],
        out_specs=[
            pl.BlockSpec((BLK, D), lambda i: (i, 0)),
            pl.BlockSpec((1, D), lambda i: (0, 0)),
        ],
        out_shape=[
            jax.ShapeDtypeStruct((N, D), jnp.float32),
            jax.ShapeDtypeStruct((1, D), jnp.float32),
        ],
    )(m_parts, din_b, w, b2d)


def kernel(feature, edge_index, W0, b0, W1, b1, W2, b2):
    src = edge_index[0]
    dst = edge_index[1]

    degp = _deg(src, dst).reshape(2, NW, N, 1)
    h0s, din_b, dout_b = _prep(degp, feature)

    m0 = _agg(h0s, src, dst)
    h1s = _layer(m0, din_b, dout_b, W0, b0.reshape(1, D))
    m1 = _agg(h1s, src, dst)
    h2s = _layer(m1, din_b, dout_b, W1, b1.reshape(1, D))
    m2 = _agg(h2s, src, dst)
    h, hg = _final(m2, din_b, W2, b2.reshape(1, D))
    return (h, hg)


# speedup vs baseline: 5.3448x; 1.2035x over previous
import dataclasses
import functools

import jax
import jax.numpy as jnp
from jax import lax
from jax.experimental import pallas as pl
from jax.experimental.pallas import tpu as pltpu
from jax.experimental.pallas import tpu_sc as plsc

N = 10000
E = 320000
D = 128

NC = 2
NS = 16
NW = NC * NS
EPW = E // NW
K = 128
NFULL = EPW // K
TAIL = EPW - NFULL * K
KA = 80
NCH = EPW // KA
RPS = N // NS
ZR = 25
WB = 624
CH = 208

BLK = 400
NBLK = N // BLK

_MESH = plsc.VectorSubcoreMesh(core_axis_name="c", subcore_axis_name="s")

_SC_PARAMS = pltpu.CompilerParams()
if "needs_layout_passes" in pltpu.CompilerParams.__dataclass_fields__:
    _SC_PARAMS = dataclasses.replace(_SC_PARAMS, needs_layout_passes=False)


@functools.partial(
    pl.kernel,
    out_type=jax.ShapeDtypeStruct((2 * NW * N,), jnp.float32),
    mesh=_MESH,
    scratch_types=[
        pltpu.VMEM((N,), jnp.float32),
        pltpu.VMEM((N,), jnp.float32),
        pltpu.VMEM((K,), jnp.int32),
        pltpu.VMEM((K,), jnp.int32),
    ],
    compiler_params=_SC_PARAMS,
)
def _deg(src_hbm, dst_hbm, out_hbm, acc_s, acc_d, idx_s, idx_d):
    c = lax.axis_index("c")
    s = lax.axis_index("s")
    wid = s * NC + c

    ones16 = jnp.full((16,), 1.0, jnp.float32)
    zero16 = jnp.zeros((16,), jnp.float32)

    @pl.loop(0, N // 16)
    def _(r):
        acc_s[pl.ds(r * 16, 16)] = zero16
        acc_d[pl.ds(r * 16, 16)] = zero16

    base = wid * EPW

    @pl.loop(0, NFULL)
    def _(j):
        off = base + j * K
        pltpu.sync_copy(src_hbm.at[pl.ds(off, K)], idx_s)
        pltpu.sync_copy(dst_hbm.at[pl.ds(off, K)], idx_d)

        @pl.loop(0, K // 16)
        def _(q):
            plsc.addupdate_scatter(acc_s, [idx_s[pl.ds(q * 16, 16)]], ones16)
            plsc.addupdate_scatter(acc_d, [idx_d[pl.ds(q * 16, 16)]], ones16)

    toff = base + NFULL * K
    pltpu.sync_copy(src_hbm.at[pl.ds(toff, TAIL)], idx_s.at[pl.ds(0, TAIL)])
    pltpu.sync_copy(dst_hbm.at[pl.ds(toff, TAIL)], idx_d.at[pl.ds(0, TAIL)])
    plsc.addupdate_scatter(acc_s, [idx_s[pl.ds(0, TAIL)]], ones16)
    plsc.addupdate_scatter(acc_d, [idx_d[pl.ds(0, TAIL)]], ones16)

    pltpu.sync_copy(acc_s, out_hbm.at[pl.ds(wid * N, N)])
    pltpu.sync_copy(acc_d, out_hbm.at[pl.ds(NW * N + wid * N, N)])


@functools.partial(
    pl.kernel,
    out_type=jax.ShapeDtypeStruct((NC, N, D), jnp.float32),
    mesh=_MESH,
    scratch_types=[
        pltpu.VMEM_SHARED((N, D), jnp.float32),
        pltpu.VMEM((2 * KA,), jnp.int32),
        pltpu.VMEM((2 * KA,), jnp.int32),
        pltpu.VMEM((2 * KA,), jnp.int32),
        pltpu.VMEM((2 * KA,), jnp.int32),
        pltpu.VMEM((KA,), jnp.int32),
        pltpu.VMEM((KA,), jnp.int32),
        pltpu.VMEM((KA, D), jnp.float32),
        pltpu.VMEM((KA, D), jnp.float32),
        pltpu.VMEM((ZR, D), jnp.float32),
        pltpu.SemaphoreType.DMA,
        pltpu.SemaphoreType.DMA,
        pltpu.SemaphoreType.DMA,
        pltpu.SemaphoreType.DMA,
    ],
)
def _agg(h_hbm, src_hbm, dst_hbm, out_hbm, acc, src_p0, src_p1, dst_p0,
         dst_p1, dq_a, dq_b, rows_a, rows_b, zbuf, sem_a, sem_b, sem_i0,
         sem_i1):
    c = lax.axis_index("c")
    s = lax.axis_index("s")
    wid = s * NC + c

    zero16 = jnp.zeros((16,), jnp.float32)

    @pl.loop(0, ZR)
    def _(r):
        @pl.loop(0, D, step=16)
        def _(col):
            zbuf[r, pl.ds(col, 16)] = zero16

    @pl.loop(0, RPS // ZR)
    def _(t):
        pltpu.sync_copy(zbuf, acc.at[pl.ds(s * RPS + t * ZR, ZR)])

    plsc.subcore_barrier()

    base = wid * EPW

    def start_pair_load(q, src_p, dst_p, sem):
        off = base + q * KA
        pltpu.async_copy(src_hbm.at[pl.ds(off, 2 * KA)], src_p, sem)
        pltpu.async_copy(dst_hbm.at[pl.ds(off, 2 * KA)], dst_p, sem)

    def wait_pair_load(src_p, dst_p, sem):
        pltpu.make_async_copy(src_hbm.at[pl.ds(0, 2 * KA)], src_p, sem).wait()
        pltpu.make_async_copy(dst_hbm.at[pl.ds(0, 2 * KA)], dst_p, sem).wait()

    def start_gather(src_p, half, rows, sem):
        pltpu.async_copy(h_hbm.at[src_p.at[pl.ds(half * KA, KA)]], rows, sem)

    def wait_gather(src_p, half, rows, sem):
        pltpu.make_async_copy(
            h_hbm.at[src_p.at[pl.ds(half * KA, KA)]], rows, sem).wait()

    def copy_dq(dst_p, half, dq):
        @pl.loop(0, KA, step=16)
        def _(i):
            dq[pl.ds(i, 16)] = dst_p[pl.ds(half * KA + i, 16)]

    def scatter(rows, dq):
        pltpu.sync_copy(rows, acc.at[dq], add=True)

    start_pair_load(0, src_p0, dst_p0, sem_i0)
    wait_pair_load(src_p0, dst_p0, sem_i0)
    start_gather(src_p0, 0, rows_a, sem_a)
    start_gather(src_p0, 1, rows_b, sem_b)
    start_pair_load(2, src_p1, dst_p1, sem_i1)

    NITER = (NCH - 5) // 4

    @pl.loop(0, NITER)
    def _(it):
        q = 4 * it
        copy_dq(dst_p0, 0, dq_a)
        copy_dq(dst_p0, 1, dq_b)
        wait_gather(src_p0, 0, rows_a, sem_a)
        scatter(rows_a, dq_a)
        wait_pair_load(src_p1, dst_p1, sem_i1)
        start_gather(src_p1, 0, rows_a, sem_a)
        wait_gather(src_p0, 1, rows_b, sem_b)
        scatter(rows_b, dq_b)
        start_gather(src_p1, 1, rows_b, sem_b)
        start_pair_load(q + 4, src_p0, dst_p0, sem_i0)
        copy_dq(dst_p1, 0, dq_a)
        copy_dq(dst_p1, 1, dq_b)
        wait_gather(src_p1, 0, rows_a, sem_a)
        scatter(rows_a, dq_a)
        wait_pair_load(src_p0, dst_p0, sem_i0)
        start_gather(src_p0, 0, rows_a, sem_a)
        wait_gather(src_p1, 1, rows_b, sem_b)
        scatter(rows_b, dq_b)
        start_gather(src_p0, 1, rows_b, sem_b)
        start_pair_load(q + 6, src_p1, dst_p1, sem_i1)

    copy_dq(dst_p0, 0, dq_a)
    copy_dq(dst_p0, 1, dq_b)
    wait_gather(src_p0, 0, rows_a, sem_a)
    scatter(rows_a, dq_a)
    wait_pair_load(src_p1, dst_p1, sem_i1)
    start_gather(src_p1, 0, rows_a, sem_a)
    wait_gather(src_p0, 1, rows_b, sem_b)
    scatter(rows_b, dq_b)
    start_gather(src_p1, 1, rows_b, sem_b)
    off_last = base + (NCH - 1) * KA
    pltpu.async_copy(src_hbm.at[pl.ds(off_last, KA)],
                     src_p0.at[pl.ds(0, KA)], sem_i0)
    pltpu.async_copy(dst_hbm.at[pl.ds(off_last, KA)],
                     dst_p0.at[pl.ds(0, KA)], sem_i0)
    copy_dq(dst_p1, 0, dq_a)
    copy_dq(dst_p1, 1, dq_b)
    wait_gather(src_p1, 0, rows_a, sem_a)
    scatter(rows_a, dq_a)
    wait_gather(src_p1, 1, rows_b, sem_b)
    scatter(rows_b, dq_b)
    pltpu.make_async_copy(src_hbm.at[pl.ds(0, KA)],
                          src_p0.at[pl.ds(0, KA)], sem_i0).wait()
    pltpu.make_async_copy(dst_hbm.at[pl.ds(0, KA)],
                          dst_p0.at[pl.ds(0, KA)], sem_i0).wait()
    start_gather(src_p0, 0, rows_a, sem_a)
    copy_dq(dst_p0, 0, dq_a)
    wait_gather(src_p0, 0, rows_a, sem_a)
    scatter(rows_a, dq_a)

    plsc.subcore_barrier()

    pltpu.sync_copy(acc.at[pl.ds(s * WB, WB)],
                    out_hbm.at[c, pl.ds(s * WB, WB)])

    @pl.when(s == NS - 1)
    def _():
        pltpu.sync_copy(acc.at[pl.ds(NS * WB, N - NS * WB)],
                        out_hbm.at[c, pl.ds(NS * WB, N - NS * WB)])


def _prep_body(degp_ref, feat_ref, h0s_ref, din_ref, dout_ref):
    dsv = jnp.sum(degp_ref[0], axis=0)
    ddv = jnp.sum(degp_ref[1], axis=0)
    iso = lax.rsqrt(jnp.maximum(dsv, 1.0))
    isi = lax.rsqrt(jnp.maximum(ddv, 1.0))
    dout_ref[...] = jnp.broadcast_to(iso, (BLK, D))
    din_ref[...] = jnp.broadcast_to(isi, (BLK, D))
    h0s_ref[...] = feat_ref[...] * iso


def _prep(degp, feature):
    return pl.pallas_call(
        _prep_body,
        grid=(NBLK,),
        in_specs=[
            pl.BlockSpec((2, NW, BLK, 1), lambda i: (0, 0, i, 0)),
            pl.BlockSpec((BLK, D), lambda i: (i, 0)),
        ],
        out_specs=[
            pl.BlockSpec((BLK, D), lambda i: (i, 0)),
            pl.BlockSpec((BLK, D), lambda i: (i, 0)),
            pl.BlockSpec((BLK, D), lambda i: (i, 0)),
        ],
        out_shape=[
            jax.ShapeDtypeStruct((N, D), jnp.float32),
            jax.ShapeDtypeStruct((N, D), jnp.float32),
            jax.ShapeDtypeStruct((N, D), jnp.float32),
        ],
    )(degp, feature)


def _layer_body(m_ref, din_ref, dout_ref, w_ref, b_ref, out_ref):
    m = (m_ref[0] + m_ref[1]) * din_ref[...]
    h = lax.dot_general(m, w_ref[...], (((1,), (0,)), ((), ())),
                        precision=lax.Precision.HIGHEST,
                        preferred_element_type=jnp.float32)
    h = jnp.maximum(h + b_ref[...], 0.0)
    out_ref[...] = h * dout_ref[...]


def _layer(m_parts, din_b, dout_b, w, b2d):
    return pl.pallas_call(
        _layer_body,
        grid=(NBLK,),
        in_specs=[
            pl.BlockSpec((NC, BLK, D), lambda i: (0, i, 0)),
            pl.BlockSpec((BLK, D), lambda i: (i, 0)),
            pl.BlockSpec((BLK, D), lambda i: (i, 0)),
            pl.BlockSpec((D, D), lambda i: (0, 0)),
            pl.BlockSpec((1, D), lambda i: (0, 0)),
        ],
        out_specs=pl.BlockSpec((BLK, D), lambda i: (i, 0)),
        out_shape=jax.ShapeDtypeStruct((N, D), jnp.float32),
    )(m_parts, din_b, dout_b, w, b2d)


def _final_body(m_ref, din_ref, w_ref, b_ref, h_ref, hg_ref):
    m = (m_ref[0] + m_ref[1]) * din_ref[...]
    h = lax.dot_general(m, w_ref[...], (((1,), (0,)), ((), ())),
                        precision=lax.Precision.HIGHEST,
                        preferred_element_type=jnp.float32)
    h = jnp.maximum(h + b_ref[...], 0.0)
    h_ref[...] = h

    @pl.when(pl.program_id(0) == 0)
    def _():
        hg_ref[...] = jnp.zeros((1, D), jnp.float32)

    hg_ref[...] += jnp.sum(h, axis=0, keepdims=True) * (1.0 / N)


def _final(m_parts, din_b, w, b2d):
    return pl.pallas_call(
        _final_body,
        grid=(NBLK,),
        in_specs=[
            pl.BlockSpec((NC, BLK, D), lambda i: (0, i, 0)),
            pl.BlockSpec((BLK, D), lambda i: (i, 0)),
            pl.BlockSpec((D, D), lambda i: (0, 0)),
            pl.BlockSpec((1, D), lambda i: (0, 0)),
        ],
        out_specs=[
            pl.BlockSpec((BLK, D), lambda i: (i, 0)),
            pl.BlockSpec((1, D), lambda i: (0, 0)),
        ],
        out_shape=[
            jax.ShapeDtypeStruct((N, D), jnp.float32),
            jax.ShapeDtypeStruct((1, D), jnp.float32),
        ],
    )(m_parts, din_b, w, b2d)


def kernel(feature, edge_index, W0, b0, W1, b1, W2, b2):
    src = edge_index[0]
    dst = edge_index[1]

    degp = _deg(src, dst).reshape(2, NW, N, 1)
    h0s, din_b, dout_b = _prep(degp, feature)

    m0 = _agg(h0s, src, dst)
    h1s = _layer(m0, din_b, dout_b, W0, b0.reshape(1, D))
    m1 = _agg(h1s, src, dst)
    h2s = _layer(m1, din_b, dout_b, W1, b1.reshape(1, D))
    m2 = _agg(h2s, src, dst)
    h, hg = _final(m2, din_b, W2, b2.reshape(1, D))
    return (h, hg)


# speedup vs baseline: 7.1531x; 1.3383x over previous
import dataclasses
import functools

import jax
import jax.numpy as jnp
from jax import lax
from jax.experimental import pallas as pl
from jax.experimental.pallas import tpu as pltpu
from jax.experimental.pallas import tpu_sc as plsc

N = 10000
E = 320000
D = 128

NC = 2
NS = 16
NW = NC * NS
EPW = E // NW
K = 128
NFULL = EPW // K
TAIL = EPW - NFULL * K
KA = 80
NCH = EPW // KA
RPS = N // NS
ZR = 25
WB = 624
CH = 208

BLK = 400
NBLK = N // BLK

_MESH = plsc.VectorSubcoreMesh(core_axis_name="c", subcore_axis_name="s")

@functools.partial(
    pl.kernel,
    out_type=jax.ShapeDtypeStruct((NC, N, D), jnp.float32),
    mesh=_MESH,
    scratch_types=[
        pltpu.VMEM_SHARED((N, D), jnp.float32),
        pltpu.VMEM((2 * KA,), jnp.int32),
        pltpu.VMEM((2 * KA,), jnp.int32),
        pltpu.VMEM((2 * KA,), jnp.int32),
        pltpu.VMEM((2 * KA,), jnp.int32),
        pltpu.VMEM((KA,), jnp.int32),
        pltpu.VMEM((KA,), jnp.int32),
        pltpu.VMEM((KA, D), jnp.float32),
        pltpu.VMEM((KA, D), jnp.float32),
        pltpu.VMEM((ZR, D), jnp.float32),
        pltpu.SemaphoreType.DMA,
        pltpu.SemaphoreType.DMA,
        pltpu.SemaphoreType.DMA,
        pltpu.SemaphoreType.DMA,
    ],
)
def _agg(h_hbm, src_hbm, dst_hbm, out_hbm, acc, src_p0, src_p1, dst_p0,
         dst_p1, dq_a, dq_b, rows_a, rows_b, zbuf, sem_a, sem_b, sem_i0,
         sem_i1):
    c = lax.axis_index("c")
    s = lax.axis_index("s")
    wid = s * NC + c

    zero16 = jnp.zeros((16,), jnp.float32)

    @pl.loop(0, ZR)
    def _(r):
        @pl.loop(0, D, step=16)
        def _(col):
            zbuf[r, pl.ds(col, 16)] = zero16

    @pl.loop(0, RPS // ZR)
    def _(t):
        pltpu.sync_copy(zbuf, acc.at[pl.ds(s * RPS + t * ZR, ZR)])

    plsc.subcore_barrier()

    base = wid * EPW

    def start_pair_load(q, src_p, dst_p, sem):
        off = base + q * KA
        pltpu.async_copy(src_hbm.at[pl.ds(off, 2 * KA)], src_p, sem)
        pltpu.async_copy(dst_hbm.at[pl.ds(off, 2 * KA)], dst_p, sem)

    def wait_pair_load(src_p, dst_p, sem):
        pltpu.make_async_copy(src_hbm.at[pl.ds(0, 2 * KA)], src_p, sem).wait()
        pltpu.make_async_copy(dst_hbm.at[pl.ds(0, 2 * KA)], dst_p, sem).wait()

    def start_gather(src_p, half, rows, sem):
        pltpu.async_copy(h_hbm.at[src_p.at[pl.ds(half * KA, KA)]], rows, sem)

    def wait_gather(src_p, half, rows, sem):
        pltpu.make_async_copy(
            h_hbm.at[src_p.at[pl.ds(half * KA, KA)]], rows, sem).wait()

    def copy_dq(dst_p, half, dq):
        @pl.loop(0, KA, step=16)
        def _(i):
            dq[pl.ds(i, 16)] = dst_p[pl.ds(half * KA + i, 16)]

    def scatter(rows, dq):
        pltpu.sync_copy(rows, acc.at[dq], add=True)

    start_pair_load(0, src_p0, dst_p0, sem_i0)
    wait_pair_load(src_p0, dst_p0, sem_i0)
    start_gather(src_p0, 0, rows_a, sem_a)
    start_gather(src_p0, 1, rows_b, sem_b)
    start_pair_load(2, src_p1, dst_p1, sem_i1)

    NITER = (NCH - 5) // 4

    @pl.loop(0, NITER)
    def _(it):
        q = 4 * it
        copy_dq(dst_p0, 0, dq_a)
        copy_dq(dst_p0, 1, dq_b)
        wait_gather(src_p0, 0, rows_a, sem_a)
        scatter(rows_a, dq_a)
        wait_pair_load(src_p1, dst_p1, sem_i1)
        start_gather(src_p1, 0, rows_a, sem_a)
        wait_gather(src_p0, 1, rows_b, sem_b)
        scatter(rows_b, dq_b)
        start_gather(src_p1, 1, rows_b, sem_b)
        start_pair_load(q + 4, src_p0, dst_p0, sem_i0)
        copy_dq(dst_p1, 0, dq_a)
        copy_dq(dst_p1, 1, dq_b)
        wait_gather(src_p1, 0, rows_a, sem_a)
        scatter(rows_a, dq_a)
        wait_pair_load(src_p0, dst_p0, sem_i0)
        start_gather(src_p0, 0, rows_a, sem_a)
        wait_gather(src_p1, 1, rows_b, sem_b)
        scatter(rows_b, dq_b)
        start_gather(src_p0, 1, rows_b, sem_b)
        start_pair_load(q + 6, src_p1, dst_p1, sem_i1)

    copy_dq(dst_p0, 0, dq_a)
    copy_dq(dst_p0, 1, dq_b)
    wait_gather(src_p0, 0, rows_a, sem_a)
    scatter(rows_a, dq_a)
    wait_pair_load(src_p1, dst_p1, sem_i1)
    start_gather(src_p1, 0, rows_a, sem_a)
    wait_gather(src_p0, 1, rows_b, sem_b)
    scatter(rows_b, dq_b)
    start_gather(src_p1, 1, rows_b, sem_b)
    off_last = base + (NCH - 1) * KA
    pltpu.async_copy(src_hbm.at[pl.ds(off_last, KA)],
                     src_p0.at[pl.ds(0, KA)], sem_i0)
    pltpu.async_copy(dst_hbm.at[pl.ds(off_last, KA)],
                     dst_p0.at[pl.ds(0, KA)], sem_i0)
    copy_dq(dst_p1, 0, dq_a)
    copy_dq(dst_p1, 1, dq_b)
    wait_gather(src_p1, 0, rows_a, sem_a)
    scatter(rows_a, dq_a)
    wait_gather(src_p1, 1, rows_b, sem_b)
    scatter(rows_b, dq_b)
    pltpu.make_async_copy(src_hbm.at[pl.ds(0, KA)],
                          src_p0.at[pl.ds(0, KA)], sem_i0).wait()
    pltpu.make_async_copy(dst_hbm.at[pl.ds(0, KA)],
                          dst_p0.at[pl.ds(0, KA)], sem_i0).wait()
    start_gather(src_p0, 0, rows_a, sem_a)
    copy_dq(dst_p0, 0, dq_a)
    wait_gather(src_p0, 0, rows_a, sem_a)
    scatter(rows_a, dq_a)

    plsc.subcore_barrier()

    pltpu.sync_copy(acc.at[pl.ds(s * WB, WB)],
                    out_hbm.at[c, pl.ds(s * WB, WB)])

    @pl.when(s == NS - 1)
    def _():
        pltpu.sync_copy(acc.at[pl.ds(NS * WB, N - NS * WB)],
                        out_hbm.at[c, pl.ds(NS * WB, N - NS * WB)])


def _prep_body(dinp_ref, doutp_ref, feat_ref, h0s_ref, din_ref, dout_ref):
    isi = lax.rsqrt(jnp.maximum(dinp_ref[0] + dinp_ref[1], 1.0))
    iso = lax.rsqrt(jnp.maximum(doutp_ref[0] + doutp_ref[1], 1.0))
    din_ref[...] = isi
    dout_ref[...] = iso
    h0s_ref[...] = feat_ref[...] * iso


def _prep(din_parts, dout_parts, feature):
    return pl.pallas_call(
        _prep_body,
        grid=(NBLK,),
        in_specs=[
            pl.BlockSpec((NC, BLK, D), lambda i: (0, i, 0)),
            pl.BlockSpec((NC, BLK, D), lambda i: (0, i, 0)),
            pl.BlockSpec((BLK, D), lambda i: (i, 0)),
        ],
        out_specs=[
            pl.BlockSpec((BLK, D), lambda i: (i, 0)),
            pl.BlockSpec((BLK, D), lambda i: (i, 0)),
            pl.BlockSpec((BLK, D), lambda i: (i, 0)),
        ],
        out_shape=[
            jax.ShapeDtypeStruct((N, D), jnp.float32),
            jax.ShapeDtypeStruct((N, D), jnp.float32),
            jax.ShapeDtypeStruct((N, D), jnp.float32),
        ],
    )(din_parts, dout_parts, feature)


def _layer_body(m_ref, din_ref, dout_ref, w_ref, b_ref, out_ref):
    m = (m_ref[0] + m_ref[1]) * din_ref[...]
    h = lax.dot_general(m, w_ref[...], (((1,), (0,)), ((), ())),
                        precision=lax.Precision.HIGHEST,
                        preferred_element_type=jnp.float32)
    h = jnp.maximum(h + b_ref[...], 0.0)
    out_ref[...] = h * dout_ref[...]


def _layer(m_parts, din_b, dout_b, w, b2d):
    return pl.pallas_call(
        _layer_body,
        grid=(NBLK,),
        in_specs=[
            pl.BlockSpec((NC, BLK, D), lambda i: (0, i, 0)),
            pl.BlockSpec((BLK, D), lambda i: (i, 0)),
            pl.BlockSpec((BLK, D), lambda i: (i, 0)),
            pl.BlockSpec((D, D), lambda i: (0, 0)),
            pl.BlockSpec((1, D), lambda i: (0, 0)),
        ],
        out_specs=pl.BlockSpec((BLK, D), lambda i: (i, 0)),
        out_shape=jax.ShapeDtypeStruct((N, D), jnp.float32),
    )(m_parts, din_b, dout_b, w, b2d)


def _final_body(m_ref, din_ref, w_ref, b_ref, h_ref, hg_ref):
    m = (m_ref[0] + m_ref[1]) * din_ref[...]
    h = lax.dot_general(m, w_ref[...], (((1,), (0,)), ((), ())),
                        precision=lax.Precision.HIGHEST,
                        preferred_element_type=jnp.float32)
    h = jnp.maximum(h + b_ref[...], 0.0)
    h_ref[...] = h

    @pl.when(pl.program_id(0) == 0)
    def _():
        hg_ref[...] = jnp.zeros((1, D), jnp.float32)

    hg_ref[...] += jnp.sum(h, axis=0, keepdims=True) * (1.0 / N)


def _final(m_parts, din_b, w, b2d):
    return pl.pallas_call(
        _final_body,
        grid=(NBLK,),
        in_specs=[
            pl.BlockSpec((NC, BLK, D), lambda i: (0, i, 0)),
            pl.BlockSpec((BLK, D), lambda i: (i, 0)),
            pl.BlockSpec((D, D), lambda i: (0, 0)),
            pl.BlockSpec((1, D), lambda i: (0, 0)),
        ],
        out_specs=[
            pl.BlockSpec((BLK, D), lambda i: (i, 0)),
            pl.BlockSpec((1, D), lambda i: (0, 0)),
        ],
        out_shape=[
            jax.ShapeDtypeStruct((N, D), jnp.float32),
            jax.ShapeDtypeStruct((1, D), jnp.float32),
        ],
    )(m_parts, din_b, w, b2d)


def kernel(feature, edge_index, W0, b0, W1, b1, W2, b2):
    src = edge_index[0]
    dst = edge_index[1]

    ones_t = jnp.ones((N, D), jnp.float32)
    din_parts = _agg(ones_t, src, dst)
    dout_parts = _agg(ones_t, dst, src)
    h0s, din_b, dout_b = _prep(din_parts, dout_parts, feature)

    m0 = _agg(h0s, src, dst)
    h1s = _layer(m0, din_b, dout_b, W0, b0.reshape(1, D))
    m1 = _agg(h1s, src, dst)
    h2s = _layer(m1, din_b, dout_b, W1, b1.reshape(1, D))
    m2 = _agg(h2s, src, dst)
    h, hg = _final(m2, din_b, W2, b2.reshape(1, D))
    return (h, hg)


# speedup vs baseline: 8.1458x; 1.1388x over previous
import dataclasses
import functools

import jax
import jax.numpy as jnp
from jax import lax
from jax.experimental import pallas as pl
from jax.experimental.pallas import tpu as pltpu
from jax.experimental.pallas import tpu_sc as plsc

N = 10000
E = 320000
D = 128

NC = 2
NS = 16
NW = NC * NS
EPW = E // NW
K = 128
NFULL = EPW // K
TAIL = EPW - NFULL * K
KA = 80
NCH = EPW // KA
RPS = N // NS
ZR = 25
WB = 624
CH = 208

BLK = 400
NBLK = N // BLK

_MESH = plsc.VectorSubcoreMesh(core_axis_name="c", subcore_axis_name="s")

@functools.partial(
    pl.kernel,
    out_type=jax.ShapeDtypeStruct((NC, N, D), jnp.float32),
    mesh=_MESH,
    scratch_types=[
        pltpu.VMEM_SHARED((N, D), jnp.float32),
        pltpu.VMEM((2 * KA,), jnp.int32),
        pltpu.VMEM((2 * KA,), jnp.int32),
        pltpu.VMEM((2 * KA,), jnp.int32),
        pltpu.VMEM((2 * KA,), jnp.int32),
        pltpu.VMEM((KA,), jnp.int32),
        pltpu.VMEM((KA,), jnp.int32),
        pltpu.VMEM((KA, D), jnp.float32),
        pltpu.VMEM((KA, D), jnp.float32),
        pltpu.VMEM((ZR, D), jnp.float32),
        pltpu.VMEM((KA, D), jnp.float32),
        pltpu.VMEM((16,), jnp.int32),
        pltpu.SemaphoreType.DMA,
        pltpu.SemaphoreType.DMA,
        pltpu.SemaphoreType.DMA,
        pltpu.SemaphoreType.DMA,
    ],
)
def _agg(h_hbm, src_hbm, dst_hbm, flag_hbm, out_hbm, acc, src_p0, src_p1,
         dst_p0, dst_p1, dq_a, dq_b, rows_a, rows_b, zbuf, ones_v, flag_s,
         sem_a, sem_b, sem_i0, sem_i1):
    c = lax.axis_index("c")
    s = lax.axis_index("s")
    wid = s * NC + c

    zero16 = jnp.zeros((16,), jnp.float32)

    @pl.loop(0, ZR)
    def _(r):
        @pl.loop(0, D, step=16)
        def _(col):
            zbuf[r, pl.ds(col, 16)] = zero16

    @pl.loop(0, RPS // ZR)
    def _(t):
        pltpu.sync_copy(zbuf, acc.at[pl.ds(s * RPS + t * ZR, ZR)])

    pltpu.sync_copy(flag_hbm, flag_s)

    ones16f = jnp.full((16,), 1.0, jnp.float32)

    @pl.loop(0, KA)
    def _(r):
        @pl.loop(0, D, step=16)
        def _(col):
            ones_v[r, pl.ds(col, 16)] = ones16f

    plsc.subcore_barrier()

    base = wid * EPW

    def start_pair_load(q, src_p, dst_p, sem):
        off = base + q * KA
        pltpu.async_copy(src_hbm.at[pl.ds(off, 2 * KA)], src_p, sem)
        pltpu.async_copy(dst_hbm.at[pl.ds(off, 2 * KA)], dst_p, sem)

    def wait_pair_load(src_p, dst_p, sem):
        pltpu.make_async_copy(src_hbm.at[pl.ds(0, 2 * KA)], src_p, sem).wait()
        pltpu.make_async_copy(dst_hbm.at[pl.ds(0, 2 * KA)], dst_p, sem).wait()

    def start_gather(src_p, half, rows, sem):
        pltpu.async_copy(h_hbm.at[src_p.at[pl.ds(half * KA, KA)]], rows, sem)

    def wait_gather(src_p, half, rows, sem):
        pltpu.make_async_copy(
            h_hbm.at[src_p.at[pl.ds(half * KA, KA)]], rows, sem).wait()

    def copy_dq(dst_p, half, dq):
        @pl.loop(0, KA, step=16)
        def _(i):
            dq[pl.ds(i, 16)] = dst_p[pl.ds(half * KA + i, 16)]

    def scatter(rows, dq):
        pltpu.sync_copy(rows, acc.at[dq], add=True)

    def start_dst_load(q, dst_p, sem):
        off = base + q * KA
        pltpu.async_copy(dst_hbm.at[pl.ds(off, 2 * KA)], dst_p, sem)

    def wait_dst_load(dst_p, sem):
        pltpu.make_async_copy(dst_hbm.at[pl.ds(0, 2 * KA)], dst_p, sem).wait()

    def _gather_pipeline():
        start_pair_load(0, src_p0, dst_p0, sem_i0)
        wait_pair_load(src_p0, dst_p0, sem_i0)
        start_gather(src_p0, 0, rows_a, sem_a)
        start_gather(src_p0, 1, rows_b, sem_b)
        start_pair_load(2, src_p1, dst_p1, sem_i1)

        NITER = (NCH - 5) // 4

        @pl.loop(0, NITER)
        def _(it):
            q = 4 * it
            copy_dq(dst_p0, 0, dq_a)
            copy_dq(dst_p0, 1, dq_b)
            wait_gather(src_p0, 0, rows_a, sem_a)
            scatter(rows_a, dq_a)
            wait_pair_load(src_p1, dst_p1, sem_i1)
            start_gather(src_p1, 0, rows_a, sem_a)
            wait_gather(src_p0, 1, rows_b, sem_b)
            scatter(rows_b, dq_b)
            start_gather(src_p1, 1, rows_b, sem_b)
            start_pair_load(q + 4, src_p0, dst_p0, sem_i0)
            copy_dq(dst_p1, 0, dq_a)
            copy_dq(dst_p1, 1, dq_b)
            wait_gather(src_p1, 0, rows_a, sem_a)
            scatter(rows_a, dq_a)
            wait_pair_load(src_p0, dst_p0, sem_i0)
            start_gather(src_p0, 0, rows_a, sem_a)
            wait_gather(src_p1, 1, rows_b, sem_b)
            scatter(rows_b, dq_b)
            start_gather(src_p0, 1, rows_b, sem_b)
            start_pair_load(q + 6, src_p1, dst_p1, sem_i1)

        copy_dq(dst_p0, 0, dq_a)
        copy_dq(dst_p0, 1, dq_b)
        wait_gather(src_p0, 0, rows_a, sem_a)
        scatter(rows_a, dq_a)
        wait_pair_load(src_p1, dst_p1, sem_i1)
        start_gather(src_p1, 0, rows_a, sem_a)
        wait_gather(src_p0, 1, rows_b, sem_b)
        scatter(rows_b, dq_b)
        start_gather(src_p1, 1, rows_b, sem_b)
        off_last = base + (NCH - 1) * KA
        pltpu.async_copy(src_hbm.at[pl.ds(off_last, KA)],
                         src_p0.at[pl.ds(0, KA)], sem_i0)
        pltpu.async_copy(dst_hbm.at[pl.ds(off_last, KA)],
                         dst_p0.at[pl.ds(0, KA)], sem_i0)
        copy_dq(dst_p1, 0, dq_a)
        copy_dq(dst_p1, 1, dq_b)
        wait_gather(src_p1, 0, rows_a, sem_a)
        scatter(rows_a, dq_a)
        wait_gather(src_p1, 1, rows_b, sem_b)
        scatter(rows_b, dq_b)
        pltpu.make_async_copy(src_hbm.at[pl.ds(0, KA)],
                              src_p0.at[pl.ds(0, KA)], sem_i0).wait()
        pltpu.make_async_copy(dst_hbm.at[pl.ds(0, KA)],
                              dst_p0.at[pl.ds(0, KA)], sem_i0).wait()
        start_gather(src_p0, 0, rows_a, sem_a)
        copy_dq(dst_p0, 0, dq_a)
        wait_gather(src_p0, 0, rows_a, sem_a)
        scatter(rows_a, dq_a)


    is_hist = flag_s[pl.ds(0, 16)][0] == 1

    @pl.when(is_hist)
    def _():
        start_dst_load(0, dst_p0, sem_i0)
        wait_dst_load(dst_p0, sem_i0)
        start_dst_load(2, dst_p1, sem_i1)

        @pl.loop(0, (NCH - 5) // 4)
        def _(it):
            q = 4 * it
            copy_dq(dst_p0, 0, dq_a)
            scatter(ones_v, dq_a)
            copy_dq(dst_p0, 1, dq_b)
            scatter(ones_v, dq_b)
            wait_dst_load(dst_p1, sem_i1)
            start_dst_load(q + 4, dst_p0, sem_i0)
            copy_dq(dst_p1, 0, dq_a)
            scatter(ones_v, dq_a)
            copy_dq(dst_p1, 1, dq_b)
            scatter(ones_v, dq_b)
            wait_dst_load(dst_p0, sem_i0)
            start_dst_load(q + 6, dst_p1, sem_i1)

        copy_dq(dst_p0, 0, dq_a)
        scatter(ones_v, dq_a)
        copy_dq(dst_p0, 1, dq_b)
        scatter(ones_v, dq_b)
        wait_dst_load(dst_p1, sem_i1)
        copy_dq(dst_p1, 0, dq_a)
        scatter(ones_v, dq_a)
        copy_dq(dst_p1, 1, dq_b)
        scatter(ones_v, dq_b)
        off_last = base + (NCH - 1) * KA
        pltpu.async_copy(dst_hbm.at[pl.ds(off_last, KA)],
                         dst_p0.at[pl.ds(0, KA)], sem_i0)
        pltpu.make_async_copy(dst_hbm.at[pl.ds(0, KA)],
                              dst_p0.at[pl.ds(0, KA)], sem_i0).wait()
        copy_dq(dst_p0, 0, dq_a)
        scatter(ones_v, dq_a)

    @pl.when(jnp.logical_not(is_hist))
    def _():
        _gather_pipeline()

    plsc.subcore_barrier()

    pltpu.sync_copy(acc.at[pl.ds(s * WB, WB)],
                    out_hbm.at[c, pl.ds(s * WB, WB)])

    @pl.when(s == NS - 1)
    def _():
        pltpu.sync_copy(acc.at[pl.ds(NS * WB, N - NS * WB)],
                        out_hbm.at[c, pl.ds(NS * WB, N - NS * WB)])


def _prep_body(dinp_ref, doutp_ref, feat_ref, h0s_ref, din_ref, dout_ref):
    isi = lax.rsqrt(jnp.maximum(dinp_ref[0] + dinp_ref[1], 1.0))
    iso = lax.rsqrt(jnp.maximum(doutp_ref[0] + doutp_ref[1], 1.0))
    din_ref[...] = isi
    dout_ref[...] = iso
    h0s_ref[...] = feat_ref[...] * iso


def _prep(din_parts, dout_parts, feature):
    return pl.pallas_call(
        _prep_body,
        grid=(NBLK,),
        in_specs=[
            pl.BlockSpec((NC, BLK, D), lambda i: (0, i, 0)),
            pl.BlockSpec((NC, BLK, D), lambda i: (0, i, 0)),
            pl.BlockSpec((BLK, D), lambda i: (i, 0)),
        ],
        out_specs=[
            pl.BlockSpec((BLK, D), lambda i: (i, 0)),
            pl.BlockSpec((BLK, D), lambda i: (i, 0)),
            pl.BlockSpec((BLK, D), lambda i: (i, 0)),
        ],
        out_shape=[
            jax.ShapeDtypeStruct((N, D), jnp.float32),
            jax.ShapeDtypeStruct((N, D), jnp.float32),
            jax.ShapeDtypeStruct((N, D), jnp.float32),
        ],
    )(din_parts, dout_parts, feature)


def _layer_body(m_ref, din_ref, dout_ref, w_ref, b_ref, out_ref):
    m = (m_ref[0] + m_ref[1]) * din_ref[...]
    h = lax.dot_general(m, w_ref[...], (((1,), (0,)), ((), ())),
                        precision=lax.Precision.HIGHEST,
                        preferred_element_type=jnp.float32)
    h = jnp.maximum(h + b_ref[...], 0.0)
    out_ref[...] = h * dout_ref[...]


def _layer(m_parts, din_b, dout_b, w, b2d):
    return pl.pallas_call(
        _layer_body,
        grid=(NBLK,),
        in_specs=[
            pl.BlockSpec((NC, BLK, D), lambda i: (0, i, 0)),
            pl.BlockSpec((BLK, D), lambda i: (i, 0)),
            pl.BlockSpec((BLK, D), lambda i: (i, 0)),
            pl.BlockSpec((D, D), lambda i: (0, 0)),
            pl.BlockSpec((1, D), lambda i: (0, 0)),
        ],
        out_specs=pl.BlockSpec((BLK, D), lambda i: (i, 0)),
        out_shape=jax.ShapeDtypeStruct((N, D), jnp.float32),
    )(m_parts, din_b, dout_b, w, b2d)


def _final_body(m_ref, din_ref, w_ref, b_ref, h_ref, hg_ref):
    m = (m_ref[0] + m_ref[1]) * din_ref[...]
    h = lax.dot_general(m, w_ref[...], (((1,), (0,)), ((), ())),
                        precision=lax.Precision.HIGHEST,
                        preferred_element_type=jnp.float32)
    h = jnp.maximum(h + b_ref[...], 0.0)
    h_ref[...] = h

    @pl.when(pl.program_id(0) == 0)
    def _():
        hg_ref[...] = jnp.zeros((1, D), jnp.float32)

    hg_ref[...] += jnp.sum(h, axis=0, keepdims=True) * (1.0 / N)


def _final(m_parts, din_b, w, b2d):
    return pl.pallas_call(
        _final_body,
        grid=(NBLK,),
        in_specs=[
            pl.BlockSpec((NC, BLK, D), lambda i: (0, i, 0)),
            pl.BlockSpec((BLK, D), lambda i: (i, 0)),
            pl.BlockSpec((D, D), lambda i: (0, 0)),
            pl.BlockSpec((1, D), lambda i: (0, 0)),
        ],
        out_specs=[
            pl.BlockSpec((BLK, D), lambda i: (i, 0)),
            pl.BlockSpec((1, D), lambda i: (0, 0)),
        ],
        out_shape=[
            jax.ShapeDtypeStruct((N, D), jnp.float32),
            jax.ShapeDtypeStruct((1, D), jnp.float32),
        ],
    )(m_parts, din_b, w, b2d)


def kernel(feature, edge_index, W0, b0, W1, b1, W2, b2):
    src = edge_index[0]
    dst = edge_index[1]

    f_hist = jnp.ones((16,), jnp.int32)
    f_agg = jnp.zeros((16,), jnp.int32)
    din_parts = _agg(feature, src, dst, f_hist)
    dout_parts = _agg(feature, dst, src, f_hist)
    h0s, din_b, dout_b = _prep(din_parts, dout_parts, feature)

    m0 = _agg(h0s, src, dst, f_agg)
    h1s = _layer(m0, din_b, dout_b, W0, b0.reshape(1, D))
    m1 = _agg(h1s, src, dst, f_agg)
    h2s = _layer(m1, din_b, dout_b, W1, b1.reshape(1, D))
    m2 = _agg(h2s, src, dst, f_agg)
    h, hg = _final(m2, din_b, W2, b2.reshape(1, D))
    return (h, hg)


# speedup vs baseline: 8.1494x; 1.0004x over previous
import functools

import jax
import jax.numpy as jnp
from jax import lax
from jax.experimental import pallas as pl
from jax.experimental.pallas import tpu as pltpu
from jax.experimental.pallas import tpu_sc as plsc

N = 10000
E = 320000
D = 128

NC = 2
NS = 16
NW = NC * NS
EPW = E // NW
KA = 80
NCH = EPW // KA
RPS = N // NS
ZR = 25
WB = 624

BLK = 400
NBLK = N // BLK

_MESH = plsc.VectorSubcoreMesh(core_axis_name="c", subcore_axis_name="s")

@functools.partial(
    pl.kernel,
    out_type=jax.ShapeDtypeStruct((NC, N, D), jnp.float32),
    mesh=_MESH,
    scratch_types=[
        pltpu.VMEM_SHARED((N, D), jnp.float32),
        pltpu.VMEM((2 * KA,), jnp.int32),
        pltpu.VMEM((2 * KA,), jnp.int32),
        pltpu.VMEM((2 * KA,), jnp.int32),
        pltpu.VMEM((2 * KA,), jnp.int32),
        pltpu.VMEM((KA,), jnp.int32),
        pltpu.VMEM((KA,), jnp.int32),
        pltpu.VMEM((KA, D), jnp.float32),
        pltpu.VMEM((KA, D), jnp.float32),
        pltpu.VMEM((ZR, D), jnp.float32),
        pltpu.VMEM((KA, D), jnp.float32),
        pltpu.VMEM((16,), jnp.int32),
        pltpu.SemaphoreType.DMA,
        pltpu.SemaphoreType.DMA,
        pltpu.SemaphoreType.DMA,
        pltpu.SemaphoreType.DMA,
    ],
)
def _agg(h_hbm, src_hbm, dst_hbm, flag_hbm, out_hbm, acc, src_p0, src_p1,
         dst_p0, dst_p1, dq_a, dq_b, rows_a, rows_b, zbuf, ones_v, flag_s,
         sem_a, sem_b, sem_i0, sem_i1):
    c = lax.axis_index("c")
    s = lax.axis_index("s")
    wid = s * NC + c

    zero16 = jnp.zeros((16,), jnp.float32)

    @pl.loop(0, ZR)
    def _(r):
        @pl.loop(0, D, step=16)
        def _(col):
            zbuf[r, pl.ds(col, 16)] = zero16

    @pl.loop(0, RPS // ZR)
    def _(t):
        pltpu.sync_copy(zbuf, acc.at[pl.ds(s * RPS + t * ZR, ZR)])

    pltpu.sync_copy(flag_hbm, flag_s)

    ones16f = jnp.full((16,), 1.0, jnp.float32)

    @pl.loop(0, KA)
    def _(r):
        @pl.loop(0, D, step=16)
        def _(col):
            ones_v[r, pl.ds(col, 16)] = ones16f

    plsc.subcore_barrier()

    base = wid * EPW

    def start_pair_load(q, src_p, dst_p, sem):
        off = base + q * KA
        pltpu.async_copy(src_hbm.at[pl.ds(off, 2 * KA)], src_p, sem)
        pltpu.async_copy(dst_hbm.at[pl.ds(off, 2 * KA)], dst_p, sem)

    def wait_pair_load(src_p, dst_p, sem):
        pltpu.make_async_copy(src_hbm.at[pl.ds(0, 2 * KA)], src_p, sem).wait()
        pltpu.make_async_copy(dst_hbm.at[pl.ds(0, 2 * KA)], dst_p, sem).wait()

    def start_gather(src_p, half, rows, sem):
        pltpu.async_copy(h_hbm.at[src_p.at[pl.ds(half * KA, KA)]], rows, sem)

    def wait_gather(src_p, half, rows, sem):
        pltpu.make_async_copy(
            h_hbm.at[src_p.at[pl.ds(half * KA, KA)]], rows, sem).wait()

    def copy_dq(dst_p, half, dq):
        @pl.loop(0, KA, step=16)
        def _(i):
            dq[pl.ds(i, 16)] = dst_p[pl.ds(half * KA + i, 16)]

    def scatter(rows, dq):
        pltpu.sync_copy(rows, acc.at[dq], add=True)

    def start_dst_load(q, dst_p, sem):
        off = base + q * KA
        pltpu.async_copy(dst_hbm.at[pl.ds(off, 2 * KA)], dst_p, sem)

    def wait_dst_load(dst_p, sem):
        pltpu.make_async_copy(dst_hbm.at[pl.ds(0, 2 * KA)], dst_p, sem).wait()

    def _gather_pipeline():
        start_pair_load(0, src_p0, dst_p0, sem_i0)
        wait_pair_load(src_p0, dst_p0, sem_i0)
        start_gather(src_p0, 0, rows_a, sem_a)
        start_gather(src_p0, 1, rows_b, sem_b)
        start_pair_load(2, src_p1, dst_p1, sem_i1)

        NITER = (NCH - 5) // 4

        @pl.loop(0, NITER)
        def _(it):
            q = 4 * it
            copy_dq(dst_p0, 0, dq_a)
            copy_dq(dst_p0, 1, dq_b)
            wait_gather(src_p0, 0, rows_a, sem_a)
            scatter(rows_a, dq_a)
            wait_pair_load(src_p1, dst_p1, sem_i1)
            start_gather(src_p1, 0, rows_a, sem_a)
            wait_gather(src_p0, 1, rows_b, sem_b)
            scatter(rows_b, dq_b)
            start_gather(src_p1, 1, rows_b, sem_b)
            start_pair_load(q + 4, src_p0, dst_p0, sem_i0)
            copy_dq(dst_p1, 0, dq_a)
            copy_dq(dst_p1, 1, dq_b)
            wait_gather(src_p1, 0, rows_a, sem_a)
            scatter(rows_a, dq_a)
            wait_pair_load(src_p0, dst_p0, sem_i0)
            start_gather(src_p0, 0, rows_a, sem_a)
            wait_gather(src_p1, 1, rows_b, sem_b)
            scatter(rows_b, dq_b)
            start_gather(src_p0, 1, rows_b, sem_b)
            start_pair_load(q + 6, src_p1, dst_p1, sem_i1)

        copy_dq(dst_p0, 0, dq_a)
        copy_dq(dst_p0, 1, dq_b)
        wait_gather(src_p0, 0, rows_a, sem_a)
        scatter(rows_a, dq_a)
        wait_pair_load(src_p1, dst_p1, sem_i1)
        start_gather(src_p1, 0, rows_a, sem_a)
        wait_gather(src_p0, 1, rows_b, sem_b)
        scatter(rows_b, dq_b)
        start_gather(src_p1, 1, rows_b, sem_b)
        off_last = base + (NCH - 1) * KA
        pltpu.async_copy(src_hbm.at[pl.ds(off_last, KA)],
                         src_p0.at[pl.ds(0, KA)], sem_i0)
        pltpu.async_copy(dst_hbm.at[pl.ds(off_last, KA)],
                         dst_p0.at[pl.ds(0, KA)], sem_i0)
        copy_dq(dst_p1, 0, dq_a)
        copy_dq(dst_p1, 1, dq_b)
        wait_gather(src_p1, 0, rows_a, sem_a)
        scatter(rows_a, dq_a)
        wait_gather(src_p1, 1, rows_b, sem_b)
        scatter(rows_b, dq_b)
        pltpu.make_async_copy(src_hbm.at[pl.ds(0, KA)],
                              src_p0.at[pl.ds(0, KA)], sem_i0).wait()
        pltpu.make_async_copy(dst_hbm.at[pl.ds(0, KA)],
                              dst_p0.at[pl.ds(0, KA)], sem_i0).wait()
        start_gather(src_p0, 0, rows_a, sem_a)
        copy_dq(dst_p0, 0, dq_a)
        wait_gather(src_p0, 0, rows_a, sem_a)
        scatter(rows_a, dq_a)


    is_hist = flag_s[pl.ds(0, 16)][0] == 1

    @pl.when(is_hist)
    def _():
        start_dst_load(0, dst_p0, sem_i0)
        wait_dst_load(dst_p0, sem_i0)
        start_dst_load(2, dst_p1, sem_i1)

        @pl.loop(0, (NCH - 5) // 4)
        def _(it):
            q = 4 * it
            copy_dq(dst_p0, 0, dq_a)
            scatter(ones_v, dq_a)
            copy_dq(dst_p0, 1, dq_b)
            scatter(ones_v, dq_b)
            wait_dst_load(dst_p1, sem_i1)
            start_dst_load(q + 4, dst_p0, sem_i0)
            copy_dq(dst_p1, 0, dq_a)
            scatter(ones_v, dq_a)
            copy_dq(dst_p1, 1, dq_b)
            scatter(ones_v, dq_b)
            wait_dst_load(dst_p0, sem_i0)
            start_dst_load(q + 6, dst_p1, sem_i1)

        copy_dq(dst_p0, 0, dq_a)
        scatter(ones_v, dq_a)
        copy_dq(dst_p0, 1, dq_b)
        scatter(ones_v, dq_b)
        wait_dst_load(dst_p1, sem_i1)
        copy_dq(dst_p1, 0, dq_a)
        scatter(ones_v, dq_a)
        copy_dq(dst_p1, 1, dq_b)
        scatter(ones_v, dq_b)
        off_last = base + (NCH - 1) * KA
        pltpu.async_copy(dst_hbm.at[pl.ds(off_last, KA)],
                         dst_p0.at[pl.ds(0, KA)], sem_i0)
        pltpu.make_async_copy(dst_hbm.at[pl.ds(0, KA)],
                              dst_p0.at[pl.ds(0, KA)], sem_i0).wait()
        copy_dq(dst_p0, 0, dq_a)
        scatter(ones_v, dq_a)

    @pl.when(jnp.logical_not(is_hist))
    def _():
        _gather_pipeline()

    plsc.subcore_barrier()

    pltpu.sync_copy(acc.at[pl.ds(s * WB, WB)],
                    out_hbm.at[c, pl.ds(s * WB, WB)])

    @pl.when(s == NS - 1)
    def _():
        pltpu.sync_copy(acc.at[pl.ds(NS * WB, N - NS * WB)],
                        out_hbm.at[c, pl.ds(NS * WB, N - NS * WB)])


def _prep_body(dinp_ref, doutp_ref, feat_ref, h0s_ref, din_ref, dout_ref):
    isi = lax.rsqrt(jnp.maximum(dinp_ref[0] + dinp_ref[1], 1.0))
    iso = lax.rsqrt(jnp.maximum(doutp_ref[0] + doutp_ref[1], 1.0))
    din_ref[...] = isi
    dout_ref[...] = iso
    h0s_ref[...] = feat_ref[...] * iso


def _prep(din_parts, dout_parts, feature):
    return pl.pallas_call(
        _prep_body,
        grid=(NBLK,),
        in_specs=[
            pl.BlockSpec((NC, BLK, D), lambda i: (0, i, 0)),
            pl.BlockSpec((NC, BLK, D), lambda i: (0, i, 0)),
            pl.BlockSpec((BLK, D), lambda i: (i, 0)),
        ],
        out_specs=[
            pl.BlockSpec((BLK, D), lambda i: (i, 0)),
            pl.BlockSpec((BLK, D), lambda i: (i, 0)),
            pl.BlockSpec((BLK, D), lambda i: (i, 0)),
        ],
        out_shape=[
            jax.ShapeDtypeStruct((N, D), jnp.float32),
            jax.ShapeDtypeStruct((N, D), jnp.float32),
            jax.ShapeDtypeStruct((N, D), jnp.float32),
        ],
    )(din_parts, dout_parts, feature)


def _layer_body(m_ref, din_ref, dout_ref, w_ref, b_ref, out_ref):
    m = (m_ref[0] + m_ref[1]) * din_ref[...]
    h = lax.dot_general(m, w_ref[...], (((1,), (0,)), ((), ())),
                        precision=lax.Precision.HIGHEST,
                        preferred_element_type=jnp.float32)
    h = jnp.maximum(h + b_ref[...], 0.0)
    out_ref[...] = h * dout_ref[...]


def _layer(m_parts, din_b, dout_b, w, b2d):
    return pl.pallas_call(
        _layer_body,
        grid=(NBLK,),
        in_specs=[
            pl.BlockSpec((NC, BLK, D), lambda i: (0, i, 0)),
            pl.BlockSpec((BLK, D), lambda i: (i, 0)),
            pl.BlockSpec((BLK, D), lambda i: (i, 0)),
            pl.BlockSpec((D, D), lambda i: (0, 0)),
            pl.BlockSpec((1, D), lambda i: (0, 0)),
        ],
        out_specs=pl.BlockSpec((BLK, D), lambda i: (i, 0)),
        out_shape=jax.ShapeDtypeStruct((N, D), jnp.float32),
    )(m_parts, din_b, dout_b, w, b2d)


def _final_body(m_ref, din_ref, w_ref, b_ref, h_ref, hg_ref):
    m = (m_ref[0] + m_ref[1]) * din_ref[...]
    h = lax.dot_general(m, w_ref[...], (((1,), (0,)), ((), ())),
                        precision=lax.Precision.HIGHEST,
                        preferred_element_type=jnp.float32)
    h = jnp.maximum(h + b_ref[...], 0.0)
    h_ref[...] = h

    @pl.when(pl.program_id(0) == 0)
    def _():
        hg_ref[...] = jnp.zeros((1, D), jnp.float32)

    hg_ref[...] += jnp.sum(h, axis=0, keepdims=True) * (1.0 / N)


def _final(m_parts, din_b, w, b2d):
    return pl.pallas_call(
        _final_body,
        grid=(NBLK,),
        in_specs=[
            pl.BlockSpec((NC, BLK, D), lambda i: (0, i, 0)),
            pl.BlockSpec((BLK, D), lambda i: (i, 0)),
            pl.BlockSpec((D, D), lambda i: (0, 0)),
            pl.BlockSpec((1, D), lambda i: (0, 0)),
        ],
        out_specs=[
            pl.BlockSpec((BLK, D), lambda i: (i, 0)),
            pl.BlockSpec((1, D), lambda i: (0, 0)),
        ],
        out_shape=[
            jax.ShapeDtypeStruct((N, D), jnp.float32),
            jax.ShapeDtypeStruct((1, D), jnp.float32),
        ],
    )(m_parts, din_b, w, b2d)


def kernel(feature, edge_index, W0, b0, W1, b1, W2, b2):
    src = edge_index[0]
    dst = edge_index[1]

    f_hist = jnp.ones((16,), jnp.int32)
    f_agg = jnp.zeros((16,), jnp.int32)
    din_parts = _agg(feature, src, dst, f_hist)
    dout_parts = _agg(feature, dst, src, f_hist)
    h0s, din_b, dout_b = _prep(din_parts, dout_parts, feature)

    m0 = _agg(h0s, src, dst, f_agg)
    h1s = _layer(m0, din_b, dout_b, W0, b0.reshape(1, D))
    m1 = _agg(h1s, src, dst, f_agg)
    h2s = _layer(m1, din_b, dout_b, W1, b1.reshape(1, D))
    m2 = _agg(h2s, src, dst, f_agg)
    h, hg = _final(m2, din_b, W2, b2.reshape(1, D))
    return (h, hg)
